# baseline (speedup 1.0000x reference)
import functools

import jax
import jax.numpy as jnp
from jax import lax
from jax.experimental import pallas as pl
from jax.experimental.pallas import tpu as pltpu
from jax.experimental.pallas import tpu_sc as plsc

NUM_GENRE_PERIOD = 24
NUM_GENRES = 1000
EMBED_SIZE = 128
BATCH = 16384
G = 50
TABLE_ROWS = NUM_GENRE_PERIOD * NUM_GENRES

NUM_WORKERS = 32
BPW = BATCH // NUM_WORKERS
LANES = 16
GROUPS = BPW // LANES

N_BLOCKS = 8
ROW_BLOCK = TABLE_ROWS // N_BLOCKS
WPB = NUM_WORKERS // N_BLOCKS


def _project_kernel(tab_ref, w_ref, b_ref, ig_ref, v_ref, igt_ref):
    s = jnp.dot(tab_ref[:, :], w_ref[:, :].T,
                preferred_element_type=jnp.float32)
    row = (lax.broadcasted_iota(jnp.int32, (ROW_BLOCK, 1), 0)
           + pl.program_id(0) * ROW_BLOCK)
    v_ref[:, :] = jnp.where(row < NUM_GENRE_PERIOD, 0.0, s) + b_ref[0, 0]
    x = ig_ref[:, :].reshape(WPB, BPW, G)
    igt_ref[:, :, :] = jnp.swapaxes(x, 1, 2)


def _project_and_relayout(embed_table, fc_w, fc_b, ig):
    return pl.pallas_call(
        _project_kernel,
        grid=(N_BLOCKS,),
        in_specs=[
            pl.BlockSpec((ROW_BLOCK, EMBED_SIZE), lambda i: (i, 0)),
            pl.BlockSpec((1, EMBED_SIZE), lambda i: (0, 0)),
            pl.BlockSpec((1, 1), lambda i: (0, 0)),
            pl.BlockSpec((WPB * BPW, G), lambda i: (i, 0)),
        ],
        out_specs=[
            pl.BlockSpec((ROW_BLOCK, 1), lambda i: (i, 0)),
            pl.BlockSpec((WPB, G, BPW), lambda i: (i, 0, 0)),
        ],
        out_shape=[
            jax.ShapeDtypeStruct((TABLE_ROWS, 1), jnp.float32),
            jax.ShapeDtypeStruct((NUM_WORKERS, G, BPW), jnp.int32),
        ],
    )(embed_table, fc_w, fc_b, ig)


@functools.partial(
    pl.kernel,
    mesh=plsc.VectorSubcoreMesh(core_axis_name="c", subcore_axis_name="s"),
    out_type=jax.ShapeDtypeStruct((BATCH,), jnp.float32),
    compiler_params=pltpu.CompilerParams(needs_layout_passes=False),
    scratch_types=[
        pltpu.VMEM((TABLE_ROWS,), jnp.float32),
        pltpu.VMEM((G, BPW), jnp.int32),
        pltpu.VMEM((BPW,), jnp.int32),
        pltpu.VMEM((BPW,), jnp.float32),
        pltpu.SemaphoreType.DMA,
        pltpu.SemaphoreType.DMA,
        pltpu.SemaphoreType.DMA,
    ],
)
def _sc_pool(v_hbm, ig_hbm, t_hbm, out_hbm, v_v, ig_v, t_v, o_v,
             sem_v, sem_ig, sem_t):
    wid = lax.axis_index("s") * 2 + lax.axis_index("c")
    base = wid * BPW
    cp_v = pltpu.async_copy(v_hbm, v_v, sem_v)
    cp_ig = pltpu.async_copy(ig_hbm.at[wid], ig_v, sem_ig)
    cp_t = pltpu.async_copy(t_hbm.at[pl.ds(base, BPW)], t_v, sem_t)
    cp_t.wait()
    cp_ig.wait()
    cp_v.wait()
    inv_g = jnp.full((LANES,), 1.0 / G, dtype=jnp.float32)
    zero = jnp.zeros((LANES,), dtype=jnp.float32)

    def group(j, carry):
        tmod = lax.rem(t_v[pl.ds(j * LANES, LANES)],
                       jnp.full((LANES,), NUM_GENRE_PERIOD, dtype=jnp.int32))
        acc = jnp.zeros((LANES,), dtype=jnp.float32)
        for g in range(G):
            genre = ig_v[g, pl.ds(j * LANES, LANES)]
            jid = genre * NUM_GENRE_PERIOD + tmod
            acc = acc + plsc.load_gather(v_v, [jid])
        o_v[pl.ds(j * LANES, LANES)] = jnp.maximum(acc * inv_g, zero)
        return carry

    lax.fori_loop(0, GROUPS, group, 0)
    pltpu.sync_copy(o_v, out_hbm.at[pl.ds(base, BPW)])


def kernel(time, item_genre, embed_table, fc_w, fc_b):
    v2d, ig = _project_and_relayout(
        embed_table, fc_w, fc_b.reshape(1, 1).astype(jnp.float32),
        item_genre.astype(jnp.int32))
    v = v2d.reshape(TABLE_ROWS)
    t = time.astype(jnp.int32)
    out = _sc_pool(v, ig, t)
    return out.reshape(BATCH, 1)


# speedup vs baseline: 1.0612x; 1.0612x over previous
import functools

import jax
import jax.numpy as jnp
from jax import lax
from jax.experimental import pallas as pl
from jax.experimental.pallas import tpu as pltpu
from jax.experimental.pallas import tpu_sc as plsc

NUM_GENRE_PERIOD = 24
NUM_GENRES = 1000
EMBED_SIZE = 128
BATCH = 16384
G = 50
TABLE_ROWS = NUM_GENRE_PERIOD * NUM_GENRES

NUM_WORKERS = 32
BPW = BATCH // NUM_WORKERS
LANES = 16
GROUPS = BPW // LANES
G_UNROLL = 10

N_BLOCKS = 4
ROW_BLOCK = TABLE_ROWS // N_BLOCKS


def _project_kernel(tab_ref, w_ref, b_ref, v_ref):
    b = b_ref[0, 0]
    s = jnp.sum(tab_ref[:, :] * w_ref[:, :], axis=1, keepdims=True)
    row = (lax.broadcasted_iota(jnp.int32, (ROW_BLOCK, 1), 0)
           + pl.program_id(0) * ROW_BLOCK)
    v_ref[:, :] = jnp.where(row < NUM_GENRE_PERIOD, 0.0, s) + b


def _project_table(embed_table, fc_w, fc_b):
    return pl.pallas_call(
        _project_kernel,
        grid=(N_BLOCKS,),
        in_specs=[
            pl.BlockSpec((ROW_BLOCK, EMBED_SIZE), lambda i: (i, 0)),
            pl.BlockSpec((1, EMBED_SIZE), lambda i: (0, 0)),
            pl.BlockSpec((1, 1), lambda i: (0, 0)),
        ],
        out_specs=pl.BlockSpec((ROW_BLOCK, 1), lambda i: (i, 0)),
        out_shape=jax.ShapeDtypeStruct((TABLE_ROWS, 1), jnp.float32),
    )(embed_table, fc_w, fc_b)


@functools.partial(
    pl.kernel,
    mesh=plsc.VectorSubcoreMesh(core_axis_name="c", subcore_axis_name="s"),
    out_type=jax.ShapeDtypeStruct((BATCH,), jnp.float32),
    compiler_params=pltpu.CompilerParams(needs_layout_passes=False),
    scratch_types=[
        pltpu.VMEM((TABLE_ROWS,), jnp.float32),
        pltpu.VMEM((G, BPW), jnp.int32),
        pltpu.VMEM((BPW,), jnp.int32),
        pltpu.VMEM((BPW,), jnp.float32),
        pltpu.SemaphoreType.DMA,
        pltpu.SemaphoreType.DMA,
        pltpu.SemaphoreType.DMA,
    ],
)
def _sc_pool(v_hbm, ig_hbm, t_hbm, out_hbm, v_v, ig_v, t_v, o_v,
             sem_v, sem_ig, sem_t):
    wid = lax.axis_index("s") * 2 + lax.axis_index("c")
    base = wid * BPW
    cp_v = pltpu.async_copy(v_hbm, v_v, sem_v)
    cp_ig = pltpu.async_copy(ig_hbm.at[wid], ig_v, sem_ig)
    cp_t = pltpu.async_copy(t_hbm.at[pl.ds(base, BPW)], t_v, sem_t)
    cp_t.wait()
    cp_ig.wait()
    cp_v.wait()
    inv_g = jnp.full((LANES,), 1.0 / G, dtype=jnp.float32)
    zero = jnp.zeros((LANES,), dtype=jnp.float32)

    def group(j, carry):
        tmod = lax.rem(t_v[pl.ds(j * LANES, LANES)],
                       jnp.full((LANES,), NUM_GENRE_PERIOD, dtype=jnp.int32))

        def genre_chunk(c, acc):
            for k in range(G_UNROLL):
                genre = ig_v[c * G_UNROLL + k, pl.ds(j * LANES, LANES)]
                jid = genre * NUM_GENRE_PERIOD + tmod
                acc = acc + plsc.load_gather(v_v, [jid])
            return acc

        acc = lax.fori_loop(0, G // G_UNROLL, genre_chunk,
                            jnp.zeros((LANES,), dtype=jnp.float32))
        o_v[pl.ds(j * LANES, LANES)] = jnp.maximum(acc * inv_g, zero)
        return carry

    lax.fori_loop(0, GROUPS, group, 0)
    pltpu.sync_copy(o_v, out_hbm.at[pl.ds(base, BPW)])


def kernel(time, item_genre, embed_table, fc_w, fc_b):
    v = _project_table(embed_table, fc_w,
                       fc_b.reshape(1, 1).astype(jnp.float32)
                       ).reshape(TABLE_ROWS)
    ig = (item_genre.astype(jnp.int32)
          .reshape(NUM_WORKERS, BPW, G)
          .transpose(0, 2, 1))
    t = time.astype(jnp.int32)
    out = _sc_pool(v, ig, t)
    return out.reshape(BATCH, 1)


# speedup vs baseline: 1.1374x; 1.0718x over previous
import functools

import jax
import jax.numpy as jnp
from jax import lax
from jax.experimental import pallas as pl
from jax.experimental.pallas import tpu as pltpu
from jax.experimental.pallas import tpu_sc as plsc

NUM_GENRE_PERIOD = 24
NUM_GENRES = 1000
EMBED_SIZE = 128
BATCH = 16384
G = 50
TABLE_ROWS = NUM_GENRE_PERIOD * NUM_GENRES

NUM_WORKERS = 32
BPW = BATCH // NUM_WORKERS
LANES = 16
GROUPS = BPW // LANES
G_UNROLL = 10

N_BLOCKS = 4
ROW_BLOCK = TABLE_ROWS // N_BLOCKS


def _project_kernel(tab_ref, w_ref, b_ref, v_ref):
    b = b_ref[0, 0]
    s = jnp.sum(tab_ref[:, :] * w_ref[:, :], axis=1, keepdims=True)
    row = (lax.broadcasted_iota(jnp.int32, (ROW_BLOCK, 1), 0)
           + pl.program_id(0) * ROW_BLOCK)
    v_ref[:, :] = jnp.where(row < NUM_GENRE_PERIOD, 0.0, s) + b


def _project_table(embed_table, fc_w, fc_b):
    return pl.pallas_call(
        _project_kernel,
        grid=(N_BLOCKS,),
        in_specs=[
            pl.BlockSpec((ROW_BLOCK, EMBED_SIZE), lambda i: (i, 0)),
            pl.BlockSpec((1, EMBED_SIZE), lambda i: (0, 0)),
            pl.BlockSpec((1, 1), lambda i: (0, 0)),
        ],
        out_specs=pl.BlockSpec((ROW_BLOCK, 1), lambda i: (i, 0)),
        out_shape=jax.ShapeDtypeStruct((TABLE_ROWS, 1), jnp.float32),
    )(embed_table, fc_w, fc_b)


@functools.partial(
    pl.kernel,
    mesh=plsc.VectorSubcoreMesh(core_axis_name="c", subcore_axis_name="s"),
    out_type=jax.ShapeDtypeStruct((BATCH,), jnp.float32),
    compiler_params=pltpu.CompilerParams(needs_layout_passes=False),
    scratch_types=[
        pltpu.VMEM((TABLE_ROWS,), jnp.float32),
        pltpu.VMEM_SHARED((TABLE_ROWS,), jnp.float32),
        pltpu.VMEM((G, BPW), jnp.int32),
        pltpu.VMEM((BPW,), jnp.int32),
        pltpu.VMEM((BPW,), jnp.float32),
        pltpu.SemaphoreType.DMA,
        pltpu.SemaphoreType.DMA,
        pltpu.SemaphoreType.DMA,
    ],
)
def _sc_pool(v_hbm, ig_hbm, t_hbm, out_hbm, v_v, v_sh, ig_v, t_v, o_v,
             sem_v, sem_ig, sem_t):
    sid = lax.axis_index("s")
    wid = sid * 2 + lax.axis_index("c")
    base = wid * BPW
    cp_ig = pltpu.async_copy(ig_hbm.at[wid], ig_v, sem_ig)
    cp_t = pltpu.async_copy(t_hbm.at[pl.ds(base, BPW)], t_v, sem_t)

    @pl.when(sid == 0)
    def _():
        pltpu.sync_copy(v_hbm, v_sh)

    plsc.subcore_barrier()
    cp_v = pltpu.async_copy(v_sh, v_v, sem_v)
    cp_t.wait()
    cp_ig.wait()
    cp_v.wait()
    inv_g = jnp.full((LANES,), 1.0 / G, dtype=jnp.float32)
    zero = jnp.zeros((LANES,), dtype=jnp.float32)

    def group(j, carry):
        tmod = lax.rem(t_v[pl.ds(j * LANES, LANES)],
                       jnp.full((LANES,), NUM_GENRE_PERIOD, dtype=jnp.int32))

        def genre_chunk(c, acc):
            for k in range(G_UNROLL):
                genre = ig_v[c * G_UNROLL + k, pl.ds(j * LANES, LANES)]
                jid = genre * NUM_GENRE_PERIOD + tmod
                acc = acc + plsc.load_gather(v_v, [jid])
            return acc

        acc = lax.fori_loop(0, G // G_UNROLL, genre_chunk,
                            jnp.zeros((LANES,), dtype=jnp.float32))
        o_v[pl.ds(j * LANES, LANES)] = jnp.maximum(acc * inv_g, zero)
        return carry

    lax.fori_loop(0, GROUPS, group, 0)
    pltpu.sync_copy(o_v, out_hbm.at[pl.ds(base, BPW)])


def kernel(time, item_genre, embed_table, fc_w, fc_b):
    v = _project_table(embed_table, fc_w,
                       fc_b.reshape(1, 1).astype(jnp.float32)
                       ).reshape(TABLE_ROWS)
    ig = (item_genre.astype(jnp.int32)
          .reshape(NUM_WORKERS, BPW, G)
          .transpose(0, 2, 1))
    t = time.astype(jnp.int32)
    out = _sc_pool(v, ig, t)
    return out.reshape(BATCH, 1)


# speedup vs baseline: 1.1752x; 1.0332x over previous
import functools

import jax
import jax.numpy as jnp
from jax import lax
from jax.experimental import pallas as pl
from jax.experimental.pallas import tpu as pltpu
from jax.experimental.pallas import tpu_sc as plsc

NUM_GENRE_PERIOD = 24
NUM_GENRES = 1000
EMBED_SIZE = 128
BATCH = 16384
G = 50
TABLE_ROWS = NUM_GENRE_PERIOD * NUM_GENRES

NUM_WORKERS = 32
BPW = BATCH // NUM_WORKERS
LANES = 16
GROUPS = BPW // LANES
G_UNROLL = 10

N_BLOCKS = 2
ROW_BLOCK = TABLE_ROWS // N_BLOCKS


def _project_kernel(tab_ref, w_ref, b_ref, v_ref):
    b = b_ref[0, 0]
    s = jnp.sum(tab_ref[:, :] * w_ref[:, :], axis=1, keepdims=True)
    row = (lax.broadcasted_iota(jnp.int32, (ROW_BLOCK, 1), 0)
           + pl.program_id(0) * ROW_BLOCK)
    v_ref[:, :] = jnp.where(row < NUM_GENRE_PERIOD, 0.0, s) + b


def _project_table(embed_table, fc_w, fc_b):
    return pl.pallas_call(
        _project_kernel,
        grid=(N_BLOCKS,),
        in_specs=[
            pl.BlockSpec((ROW_BLOCK, EMBED_SIZE), lambda i: (i, 0)),
            pl.BlockSpec((1, EMBED_SIZE), lambda i: (0, 0)),
            pl.BlockSpec((1, 1), lambda i: (0, 0)),
        ],
        out_specs=pl.BlockSpec((ROW_BLOCK, 1), lambda i: (i, 0)),
        out_shape=jax.ShapeDtypeStruct((TABLE_ROWS, 1), jnp.float32),
    )(embed_table, fc_w, fc_b)


@functools.partial(
    pl.kernel,
    mesh=plsc.VectorSubcoreMesh(core_axis_name="c", subcore_axis_name="s"),
    out_type=jax.ShapeDtypeStruct((BATCH,), jnp.float32),
    compiler_params=pltpu.CompilerParams(needs_layout_passes=False),
    scratch_types=[
        pltpu.VMEM((TABLE_ROWS,), jnp.float32),
        pltpu.VMEM_SHARED((TABLE_ROWS,), jnp.float32),
        pltpu.VMEM((G, BPW), jnp.int32),
        pltpu.VMEM((BPW,), jnp.int32),
        pltpu.VMEM((BPW,), jnp.float32),
        pltpu.SemaphoreType.DMA,
        pltpu.SemaphoreType.DMA,
        pltpu.SemaphoreType.DMA,
    ],
)
def _sc_pool(v_hbm, ig_hbm, t_hbm, out_hbm, v_v, v_sh, ig_v, t_v, o_v,
             sem_v, sem_ig, sem_t):
    sid = lax.axis_index("s")
    wid = sid * 2 + lax.axis_index("c")
    base = wid * BPW
    cp_ig = pltpu.async_copy(ig_hbm.at[wid], ig_v, sem_ig)
    cp_t = pltpu.async_copy(t_hbm.at[pl.ds(base, BPW)], t_v, sem_t)

    @pl.when(sid == 0)
    def _():
        pltpu.sync_copy(v_hbm, v_sh)

    plsc.subcore_barrier()
    cp_v = pltpu.async_copy(v_sh, v_v, sem_v)
    cp_t.wait()
    cp_ig.wait()
    cp_v.wait()
    inv_g = jnp.full((LANES,), 1.0 / G, dtype=jnp.float32)
    zero = jnp.zeros((LANES,), dtype=jnp.float32)

    def group(j, carry):
        tmod = lax.rem(t_v[pl.ds(j * LANES, LANES)],
                       jnp.full((LANES,), NUM_GENRE_PERIOD, dtype=jnp.int32))

        def genre_chunk(c, acc):
            for k in range(G_UNROLL):
                genre = ig_v[c * G_UNROLL + k, pl.ds(j * LANES, LANES)]
                jid = genre * NUM_GENRE_PERIOD + tmod
                acc = acc + plsc.load_gather(v_v, [jid])
            return acc

        acc = lax.fori_loop(0, G // G_UNROLL, genre_chunk,
                            jnp.zeros((LANES,), dtype=jnp.float32))
        o_v[pl.ds(j * LANES, LANES)] = jnp.maximum(acc * inv_g, zero)
        return carry

    lax.fori_loop(0, GROUPS, group, 0)
    pltpu.sync_copy(o_v, out_hbm.at[pl.ds(base, BPW)])


def kernel(time, item_genre, embed_table, fc_w, fc_b):
    v = _project_table(embed_table, fc_w,
                       fc_b.reshape(1, 1).astype(jnp.float32)
                       ).reshape(TABLE_ROWS)
    ig = (item_genre.astype(jnp.int32)
          .reshape(NUM_WORKERS, BPW, G)
          .transpose(0, 2, 1))
    t = time.astype(jnp.int32)
    out = _sc_pool(v, ig, t)
    return out.reshape(BATCH, 1)
